# baseline (speedup 1.0000x reference)
import functools

import jax
import jax.numpy as jnp
from jax import lax
from jax.experimental import pallas as pl
from jax.experimental.pallas import tpu as pltpu
from jax.experimental.pallas import tpu_sc as plsc

NC = 2
NS = 16
NW = NC * NS
LANES = 16
D = 128

N = 10000
E = 320000
NP = (N // (NS * 128) + 1) * (NS * 128)
RPT = NP // NS
CH = 80
HCH = CH // 2
EP = NW * CH * 128

_mesh = plsc.VectorSubcoreMesh(core_axis_name="c", subcore_axis_name="s")


@functools.partial(
    pl.kernel,
    out_type=jax.ShapeDtypeStruct((NC, NP, D), jnp.float32),
    scratch_types=[
        pltpu.VMEM((CH, 128), jnp.int32),
        pltpu.VMEM((128, D), jnp.float32),
        pltpu.VMEM_SHARED((NP, D), jnp.float32),
        pltpu.SemaphoreType.DMA,
    ],
    mesh=_mesh,
)
def _deg_kernel(dst_hbm, out_hbm, idx_v, ones_v, acc, sem):
    c = lax.axis_index("c")
    s = lax.axis_index("s")
    wid = c * NS + s

    def _fillz(i, _):
        for k in range(D // LANES):
            ones_v[i, pl.ds(k * LANES, LANES)] = jnp.zeros((LANES,), jnp.float32)
        return 0

    lax.fori_loop(0, 128, _fillz, 0)
    for t in range(RPT // 128):
        pltpu.sync_copy(ones_v, acc.at[pl.ds(s * RPT + t * 128, 128)])
    plsc.subcore_barrier()

    def _fill1(i, _):
        for k in range(D // LANES):
            ones_v[i, pl.ds(k * LANES, LANES)] = jnp.full((LANES,), 1.0, jnp.float32)
        return 0

    lax.fori_loop(0, 128, _fill1, 0)
    pltpu.sync_copy(dst_hbm.at[wid], idx_v)

    DEPTH = 8

    def _swait(_):
        pltpu.make_async_copy(ones_v, acc.at[pl.ds(0, 128)], sem).wait()

    for j in range(DEPTH):
        pltpu.async_copy(ones_v, acc.at[idx_v.at[j]], sem, add=True)

    def _chunk(i, _):
        _swait(None)
        pltpu.async_copy(ones_v, acc.at[idx_v.at[i + DEPTH]], sem, add=True)
        return 0

    lax.fori_loop(0, CH - DEPTH, _chunk, 0)
    for _ in range(DEPTH):
        _swait(None)
    plsc.subcore_barrier()
    pltpu.sync_copy(acc.at[pl.ds(s * RPT, RPT)],
                    out_hbm.at[c].at[pl.ds(s * RPT, RPT)])


@functools.partial(
    pl.kernel,
    out_type=jax.ShapeDtypeStruct((NC, NP, D), jnp.float32),
    scratch_types=[
        pltpu.VMEM((HCH, 128), jnp.int32),
        pltpu.VMEM((HCH, 128), jnp.int32),
        pltpu.VMEM((128, D), jnp.float32),
        pltpu.VMEM((128, D), jnp.float32),
        pltpu.VMEM_SHARED((NP, D), jnp.float32),
        pltpu.SemaphoreType.DMA,
        pltpu.SemaphoreType.DMA,
    ],
    mesh=_mesh,
)
def _agg_kernel(xs_hbm, src_hbm, dst_hbm, out_hbm,
                srcv, dstv, b0, b1, acc, g0, g1):
    c = lax.axis_index("c")
    s = lax.axis_index("s")
    wid = c * NS + s

    def _fillz(i, _):
        for k in range(D // LANES):
            b0[i, pl.ds(k * LANES, LANES)] = jnp.zeros((LANES,), jnp.float32)
        return 0

    lax.fori_loop(0, 128, _fillz, 0)
    for t in range(RPT // 128):
        pltpu.sync_copy(b0, acc.at[pl.ds(s * RPT + t * 128, 128)])
    plsc.subcore_barrier()

    bufs = (b0, b1)
    sems = (g0, g1)
    U = 8
    for p in range(2):
        pltpu.sync_copy(src_hbm.at[wid].at[pl.ds(p * HCH, HCH)], srcv)
        pltpu.sync_copy(dst_hbm.at[wid].at[pl.ds(p * HCH, HCH)], dstv)

        def _body(i, _):
            j = U * i
            d = [None, None]
            d[0] = pltpu.async_copy(xs_hbm.at[srcv.at[j]], b0, g0)
            d[1] = pltpu.async_copy(xs_hbm.at[srcv.at[j + 1]], b1, g1)
            for q in range(U):
                b = q % 2
                d[b].wait()
                pltpu.sync_copy(bufs[b], acc.at[dstv.at[j + q]], add=True)
                if q + 2 < U:
                    d[b] = pltpu.async_copy(
                        xs_hbm.at[srcv.at[j + q + 2]], bufs[b], sems[b])
            return 0

        lax.fori_loop(0, HCH // U, _body, 0)

    plsc.subcore_barrier()
    pltpu.sync_copy(acc.at[pl.ds(s * RPT, RPT)],
                    out_hbm.at[c].at[pl.ds(s * RPT, RPT)])


BLK = 1024
_GRID = NP // BLK


def _tc_in_body(x_ref, w_ref, degp_ref, xs_ref, dinv_ref):
    deg = degp_ref[0][:, 0:1] + degp_ref[1][:, 0:1] + 1.0
    dinv = lax.rsqrt(deg)
    h = jnp.dot(x_ref[...], w_ref[...], preferred_element_type=jnp.float32)
    xs_ref[...] = dinv * h
    dinv_ref[...] = dinv


_tc_in = pl.pallas_call(
    _tc_in_body,
    grid=(_GRID,),
    in_specs=[
        pl.BlockSpec((BLK, D), lambda i: (i, 0)),
        pl.BlockSpec((D, D), lambda i: (0, 0)),
        pl.BlockSpec((NC, BLK, D), lambda i: (0, i, 0)),
    ],
    out_specs=[
        pl.BlockSpec((BLK, D), lambda i: (i, 0)),
        pl.BlockSpec((BLK, 1), lambda i: (i, 0)),
    ],
    out_shape=[
        jax.ShapeDtypeStruct((NP, D), jnp.float32),
        jax.ShapeDtypeStruct((NP, 1), jnp.float32),
    ],
)


def _tc_mid_body(acc_ref, xs1_ref, dinv_ref, w2_ref, b1_ref, xs2_ref):
    srow = acc_ref[0] + acc_ref[1] + xs1_ref[...]
    dinv = dinv_ref[...]
    z = jnp.maximum(dinv * srow + b1_ref[...], 0.0)
    xs2_ref[...] = dinv * jnp.dot(z, w2_ref[...],
                                  preferred_element_type=jnp.float32)


_tc_mid = pl.pallas_call(
    _tc_mid_body,
    grid=(_GRID,),
    in_specs=[
        pl.BlockSpec((NC, BLK, D), lambda i: (0, i, 0)),
        pl.BlockSpec((BLK, D), lambda i: (i, 0)),
        pl.BlockSpec((BLK, 1), lambda i: (i, 0)),
        pl.BlockSpec((D, D), lambda i: (0, 0)),
        pl.BlockSpec((1, D), lambda i: (0, 0)),
    ],
    out_specs=pl.BlockSpec((BLK, D), lambda i: (i, 0)),
    out_shape=jax.ShapeDtypeStruct((NP, D), jnp.float32),
)


def _tc_out_body(acc_ref, xs2_ref, dinv_ref, b2_ref, o_ref):
    srow = acc_ref[0] + acc_ref[1] + xs2_ref[...]
    o_ref[...] = dinv_ref[...] * srow + b2_ref[...]


_tc_out = pl.pallas_call(
    _tc_out_body,
    grid=(_GRID,),
    in_specs=[
        pl.BlockSpec((NC, BLK, D), lambda i: (0, i, 0)),
        pl.BlockSpec((BLK, D), lambda i: (i, 0)),
        pl.BlockSpec((BLK, 1), lambda i: (i, 0)),
        pl.BlockSpec((1, D), lambda i: (0, 0)),
    ],
    out_specs=pl.BlockSpec((BLK, D), lambda i: (i, 0)),
    out_shape=jax.ShapeDtypeStruct((NP, D), jnp.float32),
)


def kernel(x, edge_index, W1, b1, W2, b2):
    n = x.shape[0]
    e = edge_index.shape[1]
    xp = jnp.zeros((NP, D), jnp.float32).at[:n].set(x)
    fill = jnp.full((EP - e,), NP - 1, jnp.int32)
    src = jnp.concatenate([edge_index[0], fill]).reshape(NW, CH, 128)
    dst = jnp.concatenate([edge_index[1], fill]).reshape(NW, CH, 128)

    degp = _deg_kernel(dst)
    xs1, dinv = _tc_in(xp, W1, degp)
    acc1 = _agg_kernel(xs1, src, dst)
    xs2 = _tc_mid(acc1, xs1, dinv, W2, b1.reshape(1, D))
    acc2 = _agg_kernel(xs2, src, dst)
    outp = _tc_out(acc2, xs2, dinv, b2.reshape(1, D))
    return outp[:n]


# speedup vs baseline: 3.0404x; 3.0404x over previous
import functools

import jax
import jax.numpy as jnp
from jax import lax
from jax.experimental import pallas as pl
from jax.experimental.pallas import tpu as pltpu
from jax.experimental.pallas import tpu_sc as plsc

NC = 2
NS = 16
NW = NC * NS
LANES = 16
D = 128

N = 10000
E = 320000
NP = (N // (NS * 128) + 1) * (NS * 128)
RPT = NP // NS
CH = 80
HCH = CH // 2
EP = NW * CH * 128

_mesh = plsc.VectorSubcoreMesh(core_axis_name="c", subcore_axis_name="s")


@functools.partial(
    pl.kernel,
    out_type=jax.ShapeDtypeStruct((NC, NP, D), jnp.float32),
    scratch_types=[
        pltpu.VMEM((CH, 128), jnp.int32),
        pltpu.VMEM((128, D), jnp.float32),
        pltpu.VMEM_SHARED((NP, D), jnp.float32),
        pltpu.SemaphoreType.DMA,
    ],
    mesh=_mesh,
)
def _deg_kernel(dst_hbm, out_hbm, idx_v, ones_v, acc, sem):
    c = lax.axis_index("c")
    s = lax.axis_index("s")
    wid = c * NS + s

    def _fillz(i, _):
        for k in range(D // LANES):
            ones_v[i, pl.ds(k * LANES, LANES)] = jnp.zeros((LANES,), jnp.float32)
        return 0

    lax.fori_loop(0, 128, _fillz, 0)
    for t in range(RPT // 128):
        pltpu.sync_copy(ones_v, acc.at[pl.ds(s * RPT + t * 128, 128)])
    plsc.subcore_barrier()

    def _fill1(i, _):
        for k in range(D // LANES):
            ones_v[i, pl.ds(k * LANES, LANES)] = jnp.full((LANES,), 1.0, jnp.float32)
        return 0

    lax.fori_loop(0, 128, _fill1, 0)
    pltpu.sync_copy(dst_hbm.at[wid], idx_v)

    DEPTH = 8

    def _swait(_):
        pltpu.make_async_copy(ones_v, acc.at[pl.ds(0, 128)], sem).wait()

    for j in range(DEPTH):
        pltpu.async_copy(ones_v, acc.at[idx_v.at[j]], sem, add=True)

    def _chunk(i, _):
        _swait(None)
        pltpu.async_copy(ones_v, acc.at[idx_v.at[i + DEPTH]], sem, add=True)
        return 0

    lax.fori_loop(0, CH - DEPTH, _chunk, 0)
    for _ in range(DEPTH):
        _swait(None)
    plsc.subcore_barrier()
    pltpu.sync_copy(acc.at[pl.ds(s * RPT, RPT)],
                    out_hbm.at[c].at[pl.ds(s * RPT, RPT)])


@functools.partial(
    pl.kernel,
    out_type=jax.ShapeDtypeStruct((NC, NP, D), jnp.float32),
    scratch_types=[
        pltpu.VMEM((HCH, 128), jnp.int32),
        pltpu.VMEM((HCH, 128), jnp.int32),
        pltpu.VMEM((128, D), jnp.float32),
        pltpu.VMEM((128, D), jnp.float32),
        pltpu.VMEM_SHARED((NP, D), jnp.float32),
        pltpu.SemaphoreType.DMA,
        pltpu.SemaphoreType.DMA,
    ],
    mesh=_mesh,
)
def _agg_kernel(xs_hbm, src_hbm, dst_hbm, out_hbm,
                srcv, dstv, b0, b1, acc, g0, g1):
    c = lax.axis_index("c")
    s = lax.axis_index("s")
    wid = c * NS + s

    def _fillz(i, _):
        for k in range(D // LANES):
            b0[i, pl.ds(k * LANES, LANES)] = jnp.zeros((LANES,), jnp.float32)
        return 0

    lax.fori_loop(0, 128, _fillz, 0)
    for t in range(RPT // 128):
        pltpu.sync_copy(b0, acc.at[pl.ds(s * RPT + t * 128, 128)])
    plsc.subcore_barrier()

    bufs = (b0, b1)
    sems = (g0, g1)
    U = 8
    for p in range(2):
        pltpu.sync_copy(src_hbm.at[wid].at[pl.ds(p * HCH, HCH)], srcv)
        pltpu.sync_copy(dst_hbm.at[wid].at[pl.ds(p * HCH, HCH)], dstv)

        def _body(i, _):
            j = U * i
            d = [None, None]
            d[0] = pltpu.async_copy(xs_hbm.at[srcv.at[j]], b0, g0)
            d[1] = pltpu.async_copy(xs_hbm.at[srcv.at[j + 1]], b1, g1)
            for q in range(U):
                b = q % 2
                d[b].wait()
                pltpu.sync_copy(bufs[b], acc.at[dstv.at[j + q]], add=True)
                if q + 2 < U:
                    d[b] = pltpu.async_copy(
                        xs_hbm.at[srcv.at[j + q + 2]], bufs[b], sems[b])
            return 0

        lax.fori_loop(0, HCH // U, _body, 0)

    plsc.subcore_barrier()
    pltpu.sync_copy(acc.at[pl.ds(s * RPT, RPT)],
                    out_hbm.at[c].at[pl.ds(s * RPT, RPT)])


BLK = 1024
_GRID = NP // BLK


def _tc_in_body(x_ref, w_ref, degp_ref, xs_ref, dinv_ref):
    deg = degp_ref[0][:, 0:1] + degp_ref[1][:, 0:1] + 1.0
    dinv = lax.rsqrt(deg)
    h = jnp.dot(x_ref[...], w_ref[...], preferred_element_type=jnp.float32)
    xs_ref[...] = dinv * h
    dinv_ref[...] = dinv


_tc_in = pl.pallas_call(
    _tc_in_body,
    grid=(_GRID,),
    in_specs=[
        pl.BlockSpec((BLK, D), lambda i: (i, 0)),
        pl.BlockSpec((D, D), lambda i: (0, 0)),
        pl.BlockSpec((NC, BLK, D), lambda i: (0, i, 0)),
    ],
    out_specs=[
        pl.BlockSpec((BLK, D), lambda i: (i, 0)),
        pl.BlockSpec((BLK, 1), lambda i: (i, 0)),
    ],
    out_shape=[
        jax.ShapeDtypeStruct((NP, D), jnp.float32),
        jax.ShapeDtypeStruct((NP, 1), jnp.float32),
    ],
)


def _tc_mid_body(acc_ref, xs1_ref, dinv_ref, w2_ref, b1_ref, xs2_ref):
    srow = acc_ref[0] + acc_ref[1] + xs1_ref[...]
    dinv = dinv_ref[...]
    z = jnp.maximum(dinv * srow + b1_ref[...], 0.0)
    xs2_ref[...] = dinv * jnp.dot(z, w2_ref[...],
                                  preferred_element_type=jnp.float32)


_tc_mid = pl.pallas_call(
    _tc_mid_body,
    grid=(_GRID,),
    in_specs=[
        pl.BlockSpec((NC, BLK, D), lambda i: (0, i, 0)),
        pl.BlockSpec((BLK, D), lambda i: (i, 0)),
        pl.BlockSpec((BLK, 1), lambda i: (i, 0)),
        pl.BlockSpec((D, D), lambda i: (0, 0)),
        pl.BlockSpec((1, D), lambda i: (0, 0)),
    ],
    out_specs=pl.BlockSpec((BLK, D), lambda i: (i, 0)),
    out_shape=jax.ShapeDtypeStruct((NP, D), jnp.float32),
)


def _tc_out_body(acc_ref, xs2_ref, dinv_ref, b2_ref, o_ref):
    srow = acc_ref[0] + acc_ref[1] + xs2_ref[...]
    o_ref[...] = dinv_ref[...] * srow + b2_ref[...]


_tc_out = pl.pallas_call(
    _tc_out_body,
    grid=(_GRID,),
    in_specs=[
        pl.BlockSpec((NC, BLK, D), lambda i: (0, i, 0)),
        pl.BlockSpec((BLK, D), lambda i: (i, 0)),
        pl.BlockSpec((BLK, 1), lambda i: (i, 0)),
        pl.BlockSpec((1, D), lambda i: (0, 0)),
    ],
    out_specs=pl.BlockSpec((BLK, D), lambda i: (i, 0)),
    out_shape=jax.ShapeDtypeStruct((NP, D), jnp.float32),
)


def kernel(x, edge_index, W1, b1, W2, b2):
    n = x.shape[0]
    e = edge_index.shape[1]
    xp = jnp.zeros((NP, D), jnp.float32).at[:n].set(x)
    fill = n + (jnp.arange(EP - e, dtype=jnp.int32) % (NP - n))
    src = jnp.concatenate([edge_index[0], fill]).reshape(NW, CH, 128)
    dst = jnp.concatenate([edge_index[1], fill]).reshape(NW, CH, 128)

    degp = _deg_kernel(dst)
    xs1, dinv = _tc_in(xp, W1, degp)
    acc1 = _agg_kernel(xs1, src, dst)
    xs2 = _tc_mid(acc1, xs1, dinv, W2, b1.reshape(1, D))
    acc2 = _agg_kernel(xs2, src, dst)
    outp = _tc_out(acc2, xs2, dinv, b2.reshape(1, D))
    return outp[:n]


# speedup vs baseline: 3.2207x; 1.0593x over previous
import functools

import jax
import jax.numpy as jnp
from jax import lax
from jax.experimental import pallas as pl
from jax.experimental.pallas import tpu as pltpu
from jax.experimental.pallas import tpu_sc as plsc

NC = 2
NS = 16
NW = NC * NS
LANES = 16
D = 128

N = 10000
E = 320000
NP = (N // (NS * 128) + 1) * (NS * 128)
RPT = NP // NS
CH = 80
HCH = CH // 2
EP = NW * CH * 128

_mesh = plsc.VectorSubcoreMesh(core_axis_name="c", subcore_axis_name="s")


@functools.partial(
    pl.kernel,
    out_type=jax.ShapeDtypeStruct((NC, NP, D), jnp.float32),
    scratch_types=[
        pltpu.VMEM((CH, 128), jnp.int32),
        pltpu.VMEM((128, D), jnp.float32),
        pltpu.VMEM_SHARED((NP, D), jnp.float32),
        pltpu.SemaphoreType.DMA,
    ],
    mesh=_mesh,
)
def _deg_kernel(dst_hbm, out_hbm, idx_v, ones_v, acc, sem):
    c = lax.axis_index("c")
    s = lax.axis_index("s")
    wid = c * NS + s

    def _fillz(i, _):
        for k in range(D // LANES):
            ones_v[i, pl.ds(k * LANES, LANES)] = jnp.zeros((LANES,), jnp.float32)
        return 0

    lax.fori_loop(0, 128, _fillz, 0)
    for t in range(RPT // 128):
        pltpu.sync_copy(ones_v, acc.at[pl.ds(s * RPT + t * 128, 128)])
    plsc.subcore_barrier()

    def _fill1(i, _):
        for k in range(D // LANES):
            ones_v[i, pl.ds(k * LANES, LANES)] = jnp.full((LANES,), 1.0, jnp.float32)
        return 0

    lax.fori_loop(0, 128, _fill1, 0)
    pltpu.sync_copy(dst_hbm.at[wid], idx_v)

    DEPTH = 8

    def _swait(_):
        pltpu.make_async_copy(ones_v, acc.at[pl.ds(0, 128)], sem).wait()

    for j in range(DEPTH):
        pltpu.async_copy(ones_v, acc.at[idx_v.at[j]], sem, add=True)

    def _chunk(i, _):
        _swait(None)
        pltpu.async_copy(ones_v, acc.at[idx_v.at[i + DEPTH]], sem, add=True)
        return 0

    lax.fori_loop(0, CH - DEPTH, _chunk, 0)
    for _ in range(DEPTH):
        _swait(None)
    plsc.subcore_barrier()
    pltpu.sync_copy(acc.at[pl.ds(s * RPT, RPT)],
                    out_hbm.at[c].at[pl.ds(s * RPT, RPT)])


@functools.partial(
    pl.kernel,
    out_type=jax.ShapeDtypeStruct((NC, NP, D), jnp.float32),
    scratch_types=[
        pltpu.VMEM((HCH, 128), jnp.int32),
        pltpu.VMEM((HCH, 128), jnp.int32),
        pltpu.VMEM((128, D), jnp.float32),
        pltpu.VMEM((128, D), jnp.float32),
        pltpu.VMEM_SHARED((NP, D), jnp.float32),
        pltpu.SemaphoreType.DMA,
        pltpu.SemaphoreType.DMA,
    ],
    mesh=_mesh,
)
def _agg_kernel(xs_hbm, src_hbm, dst_hbm, out_hbm,
                srcv, dstv, b0, b1, acc, g0, g1):
    c = lax.axis_index("c")
    s = lax.axis_index("s")
    wid = c * NS + s

    def _fillz(i, _):
        for k in range(D // LANES):
            b0[i, pl.ds(k * LANES, LANES)] = jnp.zeros((LANES,), jnp.float32)
        return 0

    lax.fori_loop(0, 128, _fillz, 0)
    for t in range(RPT // 128):
        pltpu.sync_copy(b0, acc.at[pl.ds(s * RPT + t * 128, 128)])
    plsc.subcore_barrier()

    def _wait_g(buf, sem):
        pltpu.make_async_copy(xs_hbm.at[pl.ds(0, 128)], buf, sem).wait()

    for p in range(2):
        pltpu.sync_copy(src_hbm.at[wid].at[pl.ds(p * HCH, HCH)], srcv)
        pltpu.sync_copy(dst_hbm.at[wid].at[pl.ds(p * HCH, HCH)], dstv)
        pltpu.async_copy(xs_hbm.at[srcv.at[0]], b0, g0)
        pltpu.async_copy(xs_hbm.at[srcv.at[1]], b1, g1)

        def _pair(i, _):
            j = 2 * i
            _wait_g(b0, g0)
            pltpu.sync_copy(b0, acc.at[dstv.at[j]], add=True)
            pltpu.async_copy(xs_hbm.at[srcv.at[j + 2]], b0, g0)
            _wait_g(b1, g1)
            pltpu.sync_copy(b1, acc.at[dstv.at[j + 1]], add=True)
            pltpu.async_copy(xs_hbm.at[srcv.at[j + 3]], b1, g1)
            return 0

        lax.fori_loop(0, HCH // 2 - 1, _pair, 0)
        _wait_g(b0, g0)
        pltpu.sync_copy(b0, acc.at[dstv.at[HCH - 2]], add=True)
        _wait_g(b1, g1)
        pltpu.sync_copy(b1, acc.at[dstv.at[HCH - 1]], add=True)

    plsc.subcore_barrier()
    pltpu.sync_copy(acc.at[pl.ds(s * RPT, RPT)],
                    out_hbm.at[c].at[pl.ds(s * RPT, RPT)])


BLK = 1024
_GRID = NP // BLK


def _tc_in_body(x_ref, w_ref, degp_ref, xs_ref, dinv_ref):
    deg = degp_ref[0][:, 0:1] + degp_ref[1][:, 0:1] + 1.0
    dinv = lax.rsqrt(deg)
    h = jnp.dot(x_ref[...], w_ref[...], preferred_element_type=jnp.float32)
    xs_ref[...] = dinv * h
    dinv_ref[...] = dinv


_tc_in = pl.pallas_call(
    _tc_in_body,
    grid=(_GRID,),
    in_specs=[
        pl.BlockSpec((BLK, D), lambda i: (i, 0)),
        pl.BlockSpec((D, D), lambda i: (0, 0)),
        pl.BlockSpec((NC, BLK, D), lambda i: (0, i, 0)),
    ],
    out_specs=[
        pl.BlockSpec((BLK, D), lambda i: (i, 0)),
        pl.BlockSpec((BLK, 1), lambda i: (i, 0)),
    ],
    out_shape=[
        jax.ShapeDtypeStruct((NP, D), jnp.float32),
        jax.ShapeDtypeStruct((NP, 1), jnp.float32),
    ],
)


def _tc_mid_body(acc_ref, xs1_ref, dinv_ref, w2_ref, b1_ref, xs2_ref):
    srow = acc_ref[0] + acc_ref[1] + xs1_ref[...]
    dinv = dinv_ref[...]
    z = jnp.maximum(dinv * srow + b1_ref[...], 0.0)
    xs2_ref[...] = dinv * jnp.dot(z, w2_ref[...],
                                  preferred_element_type=jnp.float32)


_tc_mid = pl.pallas_call(
    _tc_mid_body,
    grid=(_GRID,),
    in_specs=[
        pl.BlockSpec((NC, BLK, D), lambda i: (0, i, 0)),
        pl.BlockSpec((BLK, D), lambda i: (i, 0)),
        pl.BlockSpec((BLK, 1), lambda i: (i, 0)),
        pl.BlockSpec((D, D), lambda i: (0, 0)),
        pl.BlockSpec((1, D), lambda i: (0, 0)),
    ],
    out_specs=pl.BlockSpec((BLK, D), lambda i: (i, 0)),
    out_shape=jax.ShapeDtypeStruct((NP, D), jnp.float32),
)


def _tc_out_body(acc_ref, xs2_ref, dinv_ref, b2_ref, o_ref):
    srow = acc_ref[0] + acc_ref[1] + xs2_ref[...]
    o_ref[...] = dinv_ref[...] * srow + b2_ref[...]


_tc_out = pl.pallas_call(
    _tc_out_body,
    grid=(_GRID,),
    in_specs=[
        pl.BlockSpec((NC, BLK, D), lambda i: (0, i, 0)),
        pl.BlockSpec((BLK, D), lambda i: (i, 0)),
        pl.BlockSpec((BLK, 1), lambda i: (i, 0)),
        pl.BlockSpec((1, D), lambda i: (0, 0)),
    ],
    out_specs=pl.BlockSpec((BLK, D), lambda i: (i, 0)),
    out_shape=jax.ShapeDtypeStruct((NP, D), jnp.float32),
)


def kernel(x, edge_index, W1, b1, W2, b2):
    n = x.shape[0]
    e = edge_index.shape[1]
    xp = jnp.zeros((NP, D), jnp.float32).at[:n].set(x)
    fill = n + (jnp.arange(EP - e, dtype=jnp.int32) % (NP - n))
    src = jnp.concatenate([edge_index[0], fill]).reshape(NW, CH, 128)
    dst = jnp.concatenate([edge_index[1], fill]).reshape(NW, CH, 128)

    degp = _deg_kernel(dst)
    xs1, dinv = _tc_in(xp, W1, degp)
    acc1 = _agg_kernel(xs1, src, dst)
    xs2 = _tc_mid(acc1, xs1, dinv, W2, b1.reshape(1, D))
    acc2 = _agg_kernel(xs2, src, dst)
    outp = _tc_out(acc2, xs2, dinv, b2.reshape(1, D))
    return outp[:n]
